# initial kernel scaffold (unmeasured)
import jax
import jax.numpy as jnp
from jax import lax
from jax.experimental import pallas as pl
from jax.experimental.pallas import tpu as pltpu

Y_SIZE = 4
BLK = 512


def kernel(x, dy, gamma):
    m, d = x.shape
    del gamma

    def compute_body(x_ref, dy_ref, acc_ref):
        i = pl.program_id(0)

        @pl.when(i == 0)
        def _():
            acc_ref[...] = jnp.zeros_like(acc_ref)

        xb = x_ref[...]
        dyb = dy_ref[...]
        mu = jnp.mean(xb, axis=1, keepdims=True)
        xc = xb - mu
        var = jnp.mean(xc * xc, axis=1, keepdims=True)
        rstd = lax.rsqrt(var + 1e-5)
        xhat = xc * rstd
        dg = jnp.sum(dyb * xhat, axis=0, keepdims=True)
        db = jnp.sum(dyb, axis=0, keepdims=True)
        acc_ref[0:1, :] += dg
        acc_ref[1:2, :] += db

    partial = pl.pallas_call(
        compute_body,
        grid=(m // BLK,),
        in_specs=[
            pl.BlockSpec((BLK, d), lambda i: (i, 0)),
            pl.BlockSpec((BLK, d), lambda i: (i, 0)),
        ],
        out_specs=pl.BlockSpec((2, d), lambda i: (0, 0)),
        out_shape=jax.ShapeDtypeStruct((2, d), jnp.float32),
    )(x, dy)

    def ar_body(p_ref, out_ref, comm_ref, send_sems, recv_sems):
        my_x = lax.axis_index("x")
        my_y = lax.axis_index("y")
        my_z = lax.axis_index("z")

        barrier = pltpu.get_barrier_semaphore()
        for dist in range(1, Y_SIZE):
            pl.semaphore_signal(
                barrier,
                inc=1,
                device_id=(my_x, (my_y + dist) % Y_SIZE, my_z),
                device_id_type=pl.DeviceIdType.MESH,
            )
        pl.semaphore_wait(barrier, Y_SIZE - 1)

        sends = []
        for dist in range(1, Y_SIZE):
            rdma = pltpu.make_async_remote_copy(
                src_ref=p_ref,
                dst_ref=comm_ref.at[dist - 1],
                send_sem=send_sems.at[dist - 1],
                recv_sem=recv_sems.at[dist - 1],
                device_id=(my_x, (my_y + dist) % Y_SIZE, my_z),
                device_id_type=pl.DeviceIdType.MESH,
            )
            rdma.start()
            sends.append(rdma)

        for j in range(Y_SIZE - 1):
            recv = pltpu.make_async_remote_copy(
                src_ref=p_ref,
                dst_ref=comm_ref.at[j],
                send_sem=send_sems.at[j],
                recv_sem=recv_sems.at[j],
                device_id=(my_x, my_y, my_z),
                device_id_type=pl.DeviceIdType.MESH,
            )
            recv.wait_recv()

        out_ref[...] = p_ref[...] + comm_ref[0] + comm_ref[1] + comm_ref[2]

        for s in sends:
            s.wait_send()

    return pl.pallas_call(
        ar_body,
        out_shape=jax.ShapeDtypeStruct((2, d), jnp.float32),
        in_specs=[pl.BlockSpec(memory_space=pltpu.VMEM)],
        out_specs=pl.BlockSpec(memory_space=pltpu.VMEM),
        scratch_shapes=[
            pltpu.VMEM((Y_SIZE - 1, 2, d), jnp.float32),
            pltpu.SemaphoreType.DMA((Y_SIZE - 1,)),
            pltpu.SemaphoreType.DMA((Y_SIZE - 1,)),
        ],
        compiler_params=pltpu.CompilerParams(collective_id=0),
    )(partial)


# baseline (device time: 28845 ns/iter reference)
import jax
import jax.numpy as jnp
from jax import lax
from jax.experimental import pallas as pl
from jax.experimental.pallas import tpu as pltpu

Y_SIZE = 4
BLK = 512


def kernel(x, dy, gamma):
    m, d = x.shape
    del gamma
    nsteps = m // BLK

    def body(x_ref, dy_ref, out_ref, acc_ref, comm_ref, send_sems, recv_sems):
        i = pl.program_id(0)
        my_x = lax.axis_index("x")
        my_y = lax.axis_index("y")
        my_z = lax.axis_index("z")
        barrier = pltpu.get_barrier_semaphore()

        @pl.when(i == 0)
        def _():
            acc_ref[...] = jnp.zeros_like(acc_ref)
            for dist in range(1, Y_SIZE):
                pl.semaphore_signal(
                    barrier,
                    inc=1,
                    device_id=(my_x, (my_y + dist) % Y_SIZE, my_z),
                    device_id_type=pl.DeviceIdType.MESH,
                )

        xb = x_ref[...]
        dyb = dy_ref[...]
        mu = jnp.mean(xb, axis=1, keepdims=True)
        xc = xb - mu
        var = jnp.mean(xc * xc, axis=1, keepdims=True)
        rstd = lax.rsqrt(var + 1e-5)
        xhat = xc * rstd
        dg = jnp.sum(dyb * xhat, axis=0, keepdims=True)
        db = jnp.sum(dyb, axis=0, keepdims=True)
        acc_ref[0:1, :] += dg
        acc_ref[1:2, :] += db

        @pl.when(i == nsteps - 1)
        def _():
            pl.semaphore_wait(barrier, Y_SIZE - 1)

            sends = []
            for dist in range(1, Y_SIZE):
                rdma = pltpu.make_async_remote_copy(
                    src_ref=acc_ref,
                    dst_ref=comm_ref.at[dist - 1],
                    send_sem=send_sems.at[dist - 1],
                    recv_sem=recv_sems.at[dist - 1],
                    device_id=(my_x, (my_y + dist) % Y_SIZE, my_z),
                    device_id_type=pl.DeviceIdType.MESH,
                )
                rdma.start()
                sends.append(rdma)

            for j in range(Y_SIZE - 1):
                recv = pltpu.make_async_remote_copy(
                    src_ref=acc_ref,
                    dst_ref=comm_ref.at[j],
                    send_sem=send_sems.at[j],
                    recv_sem=recv_sems.at[j],
                    device_id=(my_x, my_y, my_z),
                    device_id_type=pl.DeviceIdType.MESH,
                )
                recv.wait_recv()

            out_ref[...] = (
                acc_ref[...] + comm_ref[0] + comm_ref[1] + comm_ref[2]
            )

            for s in sends:
                s.wait_send()

    return pl.pallas_call(
        body,
        grid=(nsteps,),
        in_specs=[
            pl.BlockSpec((BLK, d), lambda i: (i, 0)),
            pl.BlockSpec((BLK, d), lambda i: (i, 0)),
        ],
        out_specs=pl.BlockSpec((2, d), lambda i: (0, 0)),
        out_shape=jax.ShapeDtypeStruct((2, d), jnp.float32),
        scratch_shapes=[
            pltpu.VMEM((2, d), jnp.float32),
            pltpu.VMEM((Y_SIZE - 1, 2, d), jnp.float32),
            pltpu.SemaphoreType.DMA((Y_SIZE - 1,)),
            pltpu.SemaphoreType.DMA((Y_SIZE - 1,)),
        ],
        compiler_params=pltpu.CompilerParams(collective_id=0),
    )(x, dy)
